# baseline (speedup 1.0000x reference)
import functools

import jax
import jax.numpy as jnp
from jax import lax
from jax.experimental import pallas as pl
from jax.experimental.pallas import tpu as pltpu
from jax.experimental.pallas import tpu_sc as plsc

B, N, C, K = 2, 4096, 64, 16
CP = 128
TILE = 512
NEG_BIG = -1e30

_info = plsc.get_sparse_core_info()
NC, NS, L = _info.num_cores, _info.num_subcores, _info.num_lanes
NW = NC * NS
PTS_PER_W = N // NW
CHUNK = 32
GATHER = 128


def _make_topk_body(row_base):
    def _topk_idx_body(x_rows_ref, x_all_ref, idx_ref):
        xr = x_rows_ref[...]
        xa = x_all_ref[...]
        inner = lax.dot_general(xr, xa, (((1,), (1,)), ((), ())),
                                preferred_element_type=jnp.float32)
        xx_r = jnp.sum(xr * xr, axis=1, keepdims=True)
        xx_a = jnp.sum(xa * xa, axis=1).reshape(1, N)
        neg = 2.0 * inner - xx_r - xx_a
        colf = lax.broadcasted_iota(jnp.int32, (TILE, N), 1).astype(jnp.float32)
        BIGF = 1e9
        m = jnp.max(neg, axis=1, keepdims=True)
        picks = []
        for t in range(K):
            amin_f = jnp.min(jnp.where(neg == m, colf, BIGF), axis=1,
                             keepdims=True)
            picks.append(amin_f)
            if t < K - 1:
                m = jnp.max(jnp.where(neg < m, neg, NEG_BIG), axis=1,
                            keepdims=True)
        idx_f = jnp.minimum(jnp.concatenate(picks, axis=1), float(N - 1))
        idx_ref[...] = idx_f.astype(jnp.int32) + row_base
    return _topk_idx_body


def _head_body(feat0_ref, feat1_ref, w_ref, gamma_ref, beta_ref, out_ref):
    feat = jnp.concatenate([feat0_ref[...][:, :C], feat1_ref[...][:, :C]],
                           axis=0)
    w = w_ref[...]
    y = lax.dot_general(feat, w, (((1,), (1,)), ((), ())),
                        preferred_element_type=jnp.float32)
    mean = jnp.mean(y, axis=0, keepdims=True)
    var = jnp.mean(y * y, axis=0, keepdims=True) - mean * mean
    yhat = (y - mean) * lax.rsqrt(var + 1e-5)
    y = yhat * gamma_ref[...] + beta_ref[...]
    out_ref[...] = jnp.where(y >= 0, y, 0.01 * y)


def _sc_gather_maxpool(x_hbm, idx_hbm, out_hbm, idx_v, rows_v, feat_v, sem):
    wid = lax.axis_index("s") * NC + lax.axis_index("c")
    base_pt = wid * PTS_PER_W

    for t in range(PTS_PER_W // CHUNK):
        pt0 = base_pt + t * CHUNK
        pltpu.sync_copy(idx_hbm.at[pl.ds(pt0 * K, CHUNK * K)], idx_v)
        copies = []
        for g in range(CHUNK * K // GATHER):
            copies.append(pltpu.async_copy(
                x_hbm.at[idx_v.at[pl.ds(g * GATHER, GATHER)]],
                rows_v.at[pl.ds(g * GATHER, GATHER), :], sem))
        for cp in copies:
            cp.wait()

        def body(p, _):
            for c4 in range(C // L):
                acc = rows_v[p * K, pl.ds(c4 * L, L)]
                for j in range(1, K):
                    acc = jnp.maximum(acc, rows_v[p * K + j, pl.ds(c4 * L, L)])
                feat_v[p, pl.ds(c4 * L, L)] = acc
            return 0

        lax.fori_loop(0, CHUNK, body, 0, unroll=False)
        pltpu.sync_copy(feat_v, out_hbm.at[pl.ds(pt0, CHUNK)])


_sc_gather = functools.partial(
    pl.kernel,
    mesh=plsc.VectorSubcoreMesh(core_axis_name="c", subcore_axis_name="s"),
    out_type=jax.ShapeDtypeStruct((N, CP), jnp.float32),
    scratch_types=[
        pltpu.VMEM((CHUNK * K,), jnp.int32),
        pltpu.VMEM((CHUNK * K, CP), jnp.float32),
        pltpu.VMEM((CHUNK, CP), jnp.float32),
        pltpu.SemaphoreType.DMA,
    ],
)(_sc_gather_maxpool)


def _topk_call(xb, row_base):
    return pl.pallas_call(
        _make_topk_body(row_base),
        grid=(N // TILE,),
        in_specs=[
            pl.BlockSpec((TILE, C), lambda i: (i, 0)),
            pl.BlockSpec((N, C), lambda i: (0, 0)),
        ],
        out_specs=pl.BlockSpec((TILE, K), lambda i: (i, 0)),
        out_shape=jax.ShapeDtypeStruct((N, K), jnp.int32),
    )(xb, xb)


@jax.jit
def kernel(x, W, gamma, beta):
    x_pad = jnp.pad(x.reshape(B * N, C), ((0, 0), (0, CP - C)))
    feats = []
    for b in range(B):
        idx_b = _topk_call(x[b], b * N)
        feats.append(_sc_gather(x_pad, idx_b.reshape(N * K)))
    out = pl.pallas_call(
        _head_body,
        out_shape=jax.ShapeDtypeStruct((B * N, C), jnp.float32),
    )(feats[0], feats[1], W, gamma.reshape(1, C), beta.reshape(1, C))
    return out.reshape(B, N, C)
